# baseline (speedup 1.0000x reference)
import functools

import jax
import jax.numpy as jnp
from jax import lax
from jax.experimental import pallas as pl
from jax.experimental.pallas import tpu as pltpu
from jax.experimental.pallas import tpu_sc as plsc

_NB, _C, _H, _W = 4, 256, 200, 200
_N_ROIS, _NUM_INST = 256, 2
_RH, _RW = 14, 14
_SCALE = 0.25
_SR = 2
_NS = 28
_NW = 32
_RPW = _N_ROIS // _NW
_NPIX = _RH * _RW


def _f32(x):
    return x.astype(jnp.float32)


def _sc_body(table_hbm, rois_hbm, ipos_hbm, out_hbm, roi_v, ipos_v,
             yarr, xarr, wyarr, wxarr,
             idx_c0, idx_c1, idx_c2, idx_c3, w_all,
             buf_a, buf_b, stage0, stage1,
             sem_a, sem_b):
    wid = lax.axis_index("s") * 2 + lax.axis_index("c")
    pltpu.sync_copy(rois_hbm.at[pl.ds(wid * _RPW, _RPW)], roi_v)
    pltpu.sync_copy(ipos_hbm.at[pl.ds(wid * _RPW, _RPW)], ipos_v)

    lanes = lax.iota(jnp.int32, 16)
    lanes_f = _f32(lanes)
    qy = lax.shift_right_logical(lanes, 2)
    qx = lax.bitwise_and(lanes, 3)
    lane392 = lanes * (2 * _NPIX)

    def _vdiv(a, b):
        ones = jnp.ones((16,), jnp.float32)
        return (ones * a) / (ones * b)

    def build_axis(lo, size, arr_ref, warr_ref, limit):
        bin_sz = _vdiv(size, jnp.float32(_RH))
        limf = jnp.float32(limit)
        for half in range(2):
            sv = lanes + (half * 16)
            t = _f32(lax.shift_right_logical(sv, 1)) + (
                0.25 + 0.5 * _f32(lax.bitwise_and(sv, 1)))
            cs = lo + t * bin_sz
            valid = jnp.logical_and(cs > -1.0, cs < limf)
            cc = jnp.clip(cs, 0.0, limf - 1.0)
            c0 = cc.astype(jnp.int32)
            frac = cc - _f32(c0)
            c1 = jnp.minimum(c0 + 1, limit - 1)
            vf = jnp.where(valid, jnp.float32(0.5), jnp.float32(0.0))
            w0 = (1.0 - frac) * vf
            w1 = frac * vf
            msk = sv < _NS
            pos0 = 2 * sv
            plsc.store_scatter(arr_ref, [pos0], c0, mask=msk)
            plsc.store_scatter(arr_ref, [pos0 + 1], c1, mask=msk)
            plsc.store_scatter(warr_ref, [pos0], w0, mask=msk)
            plsc.store_scatter(warr_ref, [pos0 + 1], w1, mask=msk)

    def roi_body(r, carry):
        rrow = roi_v[r]
        b_i = rrow[0].astype(jnp.int32)
        ax1, ay1 = rrow[1], rrow[2]
        ax2, ay2 = rrow[3], rrow[4]
        bx1, by1 = rrow[6], rrow[7]
        bx2, by2 = rrow[8], rrow[9]
        min_x = jnp.minimum(ax1, bx1)
        min_y = jnp.minimum(ay1, by1)
        max_x = jnp.maximum(ax2, bx2)
        max_y = jnp.maximum(ay2, by2)

        x1s = min_x * _SCALE
        y1s = min_y * _SCALE
        roi_w = jnp.maximum(max_x * _SCALE - x1s, 1.0)
        roi_h = jnp.maximum(max_y * _SCALE - y1s, 1.0)

        build_axis(y1s, roi_h, yarr, wyarr, _H)
        build_axis(x1s, roi_w, xarr, wxarr, _W)

        irow = ipos_v[r]
        xlo0, ylo0, xhi0, yhi0 = irow[0], irow[1], irow[2], irow[3]
        xlo1, ylo1, xhi1, yhi1 = irow[4], irow[5], irow[6], irow[7]

        base_row = b_i * (_H * _W)

        def ph_body(ph, carry2):
            def pix(pw, idx_ref, slot):
                ybv = plsc.load_gather(yarr, [4 * ph + qy])
                wyv = plsc.load_gather(wyarr, [4 * ph + qy])
                xbv = plsc.load_gather(xarr, [4 * pw + qx])
                wxv = plsc.load_gather(wxarr, [4 * pw + qx])
                idx16 = base_row + ybv * _W + xbv
                w16 = wyv * wxv
                idx_ref[pl.ds(slot * 16, 16)] = idx16
                w_all[pl.ds(pw * 16, 16)] = w16

            def mk_build(idx_ref, start):
                def body(j, c):
                    pix(start + j, idx_ref, j)
                    return c
                return body

            lax.fori_loop(0, 4, mk_build(idx_c0, 0), 0)
            lax.fori_loop(0, 4, mk_build(idx_c1, 4), 0)
            lax.fori_loop(0, 4, mk_build(idx_c2, 8), 0)
            lax.fori_loop(0, 2, mk_build(idx_c3, 12), 0)

            my0 = jnp.logical_and(ph >= ylo0, ph < yhi0)
            my1 = jnp.logical_and(ph >= ylo1, ph < yhi1)

            def combine(pw, buf, slot):
                base = slot * 16
                m0 = _f32(jnp.logical_and(
                    my0, jnp.logical_and(pw >= xlo0, pw < xhi0)))
                m1 = _f32(jnp.logical_and(
                    my1, jnp.logical_and(pw >= xlo1, pw < xhi1)))

                wv = w_all[pl.ds(pw * 16, 16)]
                zero = jnp.zeros((16,), jnp.float32)
                acc_e = [zero] * (_C // 32)
                acc_o = [zero] * (_C // 32)
                for k in range(16):
                    wk = wv[k]
                    for j in range(_C // 32):
                        v = plsc.bitcast(
                            buf[base + k, pl.ds(j * 16, 16)], jnp.bfloat16)
                        e, o = plsc.unpack(
                            v, format=plsc.PackFormat.INTERLEAVED,
                            preferred_element_type=jnp.float32)
                        acc_e[j] = acc_e[j] + wk * e
                        acc_o[j] = acc_o[j] + wk * o
                pixg = ph * _RW + pw
                for j in range(_C // 32):
                    idx_e = (32 * j) * _NPIX + lane392 + pixg
                    idx_o = idx_e + _NPIX
                    plsc.store_scatter(stage0, [idx_e], acc_e[j] * m0)
                    plsc.store_scatter(stage0, [idx_o], acc_o[j] * m0)
                    plsc.store_scatter(stage1, [idx_e], acc_e[j] * m1)
                    plsc.store_scatter(stage1, [idx_o], acc_o[j] * m1)

            def mk_comb(buf, start, n):
                def body(j, c):
                    combine(start + j, buf, j)
                    return c
                return lambda: lax.fori_loop(0, n, body, 0)

            cp0 = pltpu.async_copy(table_hbm.at[idx_c0], buf_a, sem_a)
            cp1 = pltpu.async_copy(table_hbm.at[idx_c1], buf_b, sem_b)
            cp0.wait()
            mk_comb(buf_a, 0, 4)()
            cp2 = pltpu.async_copy(table_hbm.at[idx_c2], buf_a, sem_a)
            cp1.wait()
            mk_comb(buf_b, 4, 4)()
            cp3 = pltpu.async_copy(
                table_hbm.at[idx_c3], buf_b.at[pl.ds(0, 32)], sem_b)
            cp2.wait()
            mk_comb(buf_a, 8, 4)()
            cp3.wait()
            mk_comb(buf_b, 12, 2)()
            return carry2

        lax.fori_loop(0, _RH, ph_body, 0)
        roi = wid * _RPW + r
        pltpu.sync_copy(stage0, out_hbm.at[roi, 0])
        pltpu.sync_copy(stage1, out_hbm.at[roi, 1])
        return carry

    lax.fori_loop(0, _RPW, roi_body, 0)


@jax.jit
def _run(table, rois_flat, ipos):
    mesh = plsc.VectorSubcoreMesh(core_axis_name="c", subcore_axis_name="s")
    fn = functools.partial(
        pl.kernel,
        out_type=jax.ShapeDtypeStruct(
            (_N_ROIS, _NUM_INST, _C * _NPIX), jnp.float32),
        mesh=mesh,
        scratch_types=[
            pltpu.VMEM((_RPW, 16), jnp.float32),
            pltpu.VMEM((_RPW, 16), jnp.int32),
            pltpu.VMEM((64,), jnp.int32),
            pltpu.VMEM((64,), jnp.int32),
            pltpu.VMEM((64,), jnp.float32),
            pltpu.VMEM((64,), jnp.float32),
            pltpu.VMEM((64,), jnp.int32),
            pltpu.VMEM((64,), jnp.int32),
            pltpu.VMEM((64,), jnp.int32),
            pltpu.VMEM((32,), jnp.int32),
            pltpu.VMEM((224,), jnp.float32),
            pltpu.VMEM((64, _C // 2), jnp.int32),
            pltpu.VMEM((64, _C // 2), jnp.int32),
            pltpu.VMEM((_C * _NPIX,), jnp.float32),
            pltpu.VMEM((_C * _NPIX,), jnp.float32),
            pltpu.SemaphoreType.DMA,
            pltpu.SemaphoreType.DMA,
        ],
        compiler_params=pltpu.CompilerParams(needs_layout_passes=False),
    )(_sc_body)
    return fn(table, rois_flat, ipos)


def kernel(feature_maps, rois):
    table = jnp.transpose(feature_maps, (0, 2, 3, 1)).reshape(
        _NB * _H * _W, _C).astype(jnp.bfloat16)
    table = jax.lax.bitcast_convert_type(
        table.reshape(_NB * _H * _W, _C // 2, 2), jnp.int32)
    rois_flat = rois.reshape(_N_ROIS, _NUM_INST * 5)
    rois_flat = jnp.pad(rois_flat, ((0, 0), (0, 6)))
    min_x = rois[:, :, 1].min(axis=1)
    min_y = rois[:, :, 2].min(axis=1)
    max_x = rois[:, :, 3].max(axis=1)
    max_y = rois[:, :, 4].max(axis=1)
    brois_w = max_x - min_x
    brois_h = max_y - min_y
    h_ratio = _RH / brois_h
    w_ratio = _RW / brois_w
    ratios = jnp.stack(
        [w_ratio, h_ratio, w_ratio, h_ratio], axis=1).reshape(-1, 1, 4)
    bounding_lt = jnp.tile(
        jnp.stack([min_x, min_y], axis=1)[:, None, :], (1, 1, 2))
    ins_pos = ((rois[:, :, 1:] - bounding_lt) * ratios).astype(jnp.int32)
    ipos = jnp.pad(ins_pos.reshape(_N_ROIS, 8), ((0, 0), (0, 8)))
    out = _run(table, rois_flat, ipos)
    return out.reshape(_N_ROIS, _NUM_INST * _C, _RH, _RW)


# speedup vs baseline: 1.0481x; 1.0481x over previous
import functools

import jax
import jax.numpy as jnp
from jax import lax
from jax.experimental import pallas as pl
from jax.experimental.pallas import tpu as pltpu
from jax.experimental.pallas import tpu_sc as plsc

_NB, _C, _H, _W = 4, 256, 200, 200
_N_ROIS, _NUM_INST = 256, 2
_RH, _RW = 14, 14
_SCALE = 0.25
_SR = 2
_NS = 28
_NW = 32
_RPW = _N_ROIS // _NW


def _f32(x):
    return x.astype(jnp.float32)


def _sc_body(table_hbm, rois_hbm, ipos_hbm, out_hbm, roi_v, ipos_v,
             yarr, xarr, wyarr, wxarr,
             idx_a, idx_b, w_a, w_b, buf_a, buf_b, outb,
             sem_a, sem_b):
    wid = lax.axis_index("s") * 2 + lax.axis_index("c")
    pltpu.sync_copy(rois_hbm.at[pl.ds(wid * _RPW, _RPW)], roi_v)
    pltpu.sync_copy(ipos_hbm.at[pl.ds(wid * _RPW, _RPW)], ipos_v)

    lanes = lax.iota(jnp.int32, 16)
    lanes_f = _f32(lanes)
    qy = lax.shift_right_logical(lanes, 2)
    qx = lax.bitwise_and(lanes, 3)

    def _vdiv(a, b):
        ones = jnp.ones((16,), jnp.float32)
        return (ones * a) / (ones * b)

    def build_axis(lo, size, arr_ref, warr_ref, limit):
        bin_sz = _vdiv(size, jnp.float32(_RH))
        limf = jnp.float32(limit)
        for half in range(2):
            sv = lanes + (half * 16)
            svf = lanes_f + jnp.float32(half * 16)
            t = _f32(lax.shift_right_logical(sv, 1)) + (
                0.25 + 0.5 * _f32(lax.bitwise_and(sv, 1)))
            cs = lo + t * bin_sz
            valid = jnp.logical_and(cs > -1.0, cs < limf)
            cc = jnp.clip(cs, 0.0, limf - 1.0)
            c0 = cc.astype(jnp.int32)
            frac = cc - _f32(c0)
            c1 = jnp.minimum(c0 + 1, limit - 1)
            vf = jnp.where(valid, jnp.float32(0.5), jnp.float32(0.0))
            w0 = (1.0 - frac) * vf
            w1 = frac * vf
            msk = sv < _NS
            pos0 = 2 * sv
            plsc.store_scatter(arr_ref, [pos0], c0, mask=msk)
            plsc.store_scatter(arr_ref, [pos0 + 1], c1, mask=msk)
            plsc.store_scatter(warr_ref, [pos0], w0, mask=msk)
            plsc.store_scatter(warr_ref, [pos0 + 1], w1, mask=msk)

    def roi_body(r, carry):
        rrow = roi_v[r]
        b_i = rrow[0].astype(jnp.int32)
        ax1, ay1 = rrow[1], rrow[2]
        ax2, ay2 = rrow[3], rrow[4]
        bx1, by1 = rrow[6], rrow[7]
        bx2, by2 = rrow[8], rrow[9]
        min_x = jnp.minimum(ax1, bx1)
        min_y = jnp.minimum(ay1, by1)
        max_x = jnp.maximum(ax2, bx2)
        max_y = jnp.maximum(ay2, by2)

        x1s = min_x * _SCALE
        y1s = min_y * _SCALE
        roi_w = jnp.maximum(max_x * _SCALE - x1s, 1.0)
        roi_h = jnp.maximum(max_y * _SCALE - y1s, 1.0)

        build_axis(y1s, roi_h, yarr, wyarr, _H)
        build_axis(x1s, roi_w, xarr, wxarr, _W)

        irow = ipos_v[r]
        xlo0, ylo0, xhi0, yhi0 = irow[0], irow[1], irow[2], irow[3]
        xlo1, ylo1, xhi1, yhi1 = irow[4], irow[5], irow[6], irow[7]

        base_row = b_i * (_H * _W)

        def ph_body(ph, carry2):
            def pix(pw, idx_ref, w_ref, slot):
                ybv = plsc.load_gather(yarr, [4 * ph + qy])
                wyv = plsc.load_gather(wyarr, [4 * ph + qy])
                xbv = plsc.load_gather(xarr, [4 * pw + qx])
                wxv = plsc.load_gather(wxarr, [4 * pw + qx])
                idx16 = base_row + ybv * _W + xbv
                w16 = wyv * wxv
                idx_ref[pl.ds(slot * 16, 16)] = idx16
                w_ref[pl.ds(slot * 16, 16)] = w16

            def build_a(j, c):
                pix(j, idx_a, w_a, j)
                return c

            def build_b(j, c):
                pix(7 + j, idx_b, w_b, j)
                return c

            lax.fori_loop(0, 7, build_a, 0)
            lax.fori_loop(0, 7, build_b, 0)

            cp_a = pltpu.async_copy(table_hbm.at[idx_a], buf_a, sem_a)
            cp_b = pltpu.async_copy(table_hbm.at[idx_b], buf_b, sem_b)

            my0 = jnp.logical_and(ph >= ylo0, ph < yhi0)
            my1 = jnp.logical_and(ph >= ylo1, ph < yhi1)

            def combine(pw, buf, w_ref, slot):
                base = slot * 16
                m0 = _f32(jnp.logical_and(
                    my0, jnp.logical_and(pw >= xlo0, pw < xhi0)))
                m1 = _f32(jnp.logical_and(
                    my1, jnp.logical_and(pw >= xlo1, pw < xhi1)))

                wv = w_ref[pl.ds(base, 16)]
                zero = jnp.zeros((16,), jnp.float32)
                acc_e = [zero] * (_C // 32)
                acc_o = [zero] * (_C // 32)
                for k in range(16):
                    wk = wv[k]
                    for j in range(_C // 32):
                        v = plsc.bitcast(
                            buf[base + k, pl.ds(j * 16, 16)], jnp.bfloat16)
                        e, o = plsc.unpack(
                            v, format=plsc.PackFormat.INTERLEAVED,
                            preferred_element_type=jnp.float32)
                        acc_e[j] = acc_e[j] + wk * e
                        acc_o[j] = acc_o[j] + wk * o
                for j in range(_C // 32):
                    outb[0, pw, 0, pl.ds(j * 16, 16)] = acc_e[j] * m0
                    outb[0, pw, 1, pl.ds(j * 16, 16)] = acc_o[j] * m0
                    outb[1, pw, 0, pl.ds(j * 16, 16)] = acc_e[j] * m1
                    outb[1, pw, 1, pl.ds(j * 16, 16)] = acc_o[j] * m1

            cp_a.wait()

            def comb_a(j, c):
                combine(j, buf_a, w_a, j)
                return c

            lax.fori_loop(0, 7, comb_a, 0)
            cp_b.wait()

            def comb_b(j, c):
                combine(7 + j, buf_b, w_b, j)
                return c

            lax.fori_loop(0, 7, comb_b, 0)

            pltpu.sync_copy(outb, out_hbm.at[wid * _RPW + r, ph])
            return carry2

        lax.fori_loop(0, _RH, ph_body, 0)
        return carry

    lax.fori_loop(0, _RPW, roi_body, 0)


@jax.jit
def _run(table, rois_flat, ipos):
    mesh = plsc.VectorSubcoreMesh(core_axis_name="c", subcore_axis_name="s")
    fn = functools.partial(
        pl.kernel,
        out_type=jax.ShapeDtypeStruct(
            (_N_ROIS, _RH, _NUM_INST, _RW, 2, _C // 2), jnp.float32),
        mesh=mesh,
        scratch_types=[
            pltpu.VMEM((_RPW, 16), jnp.float32),
            pltpu.VMEM((_RPW, 16), jnp.int32),
            pltpu.VMEM((64,), jnp.int32),
            pltpu.VMEM((64,), jnp.int32),
            pltpu.VMEM((64,), jnp.float32),
            pltpu.VMEM((64,), jnp.float32),
            pltpu.VMEM((112,), jnp.int32),
            pltpu.VMEM((112,), jnp.int32),
            pltpu.VMEM((112,), jnp.float32),
            pltpu.VMEM((112,), jnp.float32),
            pltpu.VMEM((112, _C // 2), jnp.int32),
            pltpu.VMEM((112, _C // 2), jnp.int32),
            pltpu.VMEM((_NUM_INST, _RW, 2, _C // 2), jnp.float32),
            pltpu.SemaphoreType.DMA,
            pltpu.SemaphoreType.DMA,
        ],
        compiler_params=pltpu.CompilerParams(needs_layout_passes=False),
    )(_sc_body)
    return fn(table, rois_flat, ipos)


def kernel(feature_maps, rois):
    table = jnp.transpose(feature_maps, (0, 2, 3, 1)).reshape(
        _NB * _H * _W, _C).astype(jnp.bfloat16)
    table = jax.lax.bitcast_convert_type(
        table.reshape(_NB * _H * _W, _C // 2, 2), jnp.int32)
    rois_flat = rois.reshape(_N_ROIS, _NUM_INST * 5)
    rois_flat = jnp.pad(rois_flat, ((0, 0), (0, 6)))
    min_x = rois[:, :, 1].min(axis=1)
    min_y = rois[:, :, 2].min(axis=1)
    max_x = rois[:, :, 3].max(axis=1)
    max_y = rois[:, :, 4].max(axis=1)
    brois_w = max_x - min_x
    brois_h = max_y - min_y
    h_ratio = _RH / brois_h
    w_ratio = _RW / brois_w
    ratios = jnp.stack(
        [w_ratio, h_ratio, w_ratio, h_ratio], axis=1).reshape(-1, 1, 4)
    bounding_lt = jnp.tile(
        jnp.stack([min_x, min_y], axis=1)[:, None, :], (1, 1, 2))
    ins_pos = ((rois[:, :, 1:] - bounding_lt) * ratios).astype(jnp.int32)
    ipos = jnp.pad(ins_pos.reshape(_N_ROIS, 8), ((0, 0), (0, 8)))
    out6 = _run(table, rois_flat, ipos)
    return jnp.transpose(out6, (0, 2, 5, 4, 1, 3)).reshape(
        _N_ROIS, _NUM_INST * _C, _RH, _RW)


# speedup vs baseline: 2.4095x; 2.2989x over previous
import functools

import jax
import jax.numpy as jnp
from jax import lax
from jax.experimental import pallas as pl
from jax.experimental.pallas import tpu as pltpu
from jax.experimental.pallas import tpu_sc as plsc

_NB, _C, _H, _W = 4, 256, 200, 200
_N_ROIS, _NUM_INST = 256, 2
_RH, _RW = 14, 14
_SCALE = 0.25
_SR = 2
_NS = 28
_NW = 32
_RPW = _N_ROIS // _NW


def _f32(x):
    return x.astype(jnp.float32)


def _sc_body(table_hbm, rois_hbm, ipos_hbm, out_hbm, roi_v, ipos_v,
             yarr, xarr, wyarr, wxarr,
             idx_a, idx_b, w_a, w_b, buf_a, buf_b, outb,
             sem_a, sem_b, sem_o):
    wid = lax.axis_index("s") * 2 + lax.axis_index("c")
    pltpu.sync_copy(rois_hbm.at[pl.ds(wid * _RPW, _RPW)], roi_v)
    pltpu.sync_copy(ipos_hbm.at[pl.ds(wid * _RPW, _RPW)], ipos_v)

    lanes = lax.iota(jnp.int32, 16)
    lanes_f = _f32(lanes)
    qy = lax.shift_right_logical(lanes, 2)
    qx = lax.bitwise_and(lanes, 3)

    def _vdiv(a, b):
        ones = jnp.ones((16,), jnp.float32)
        return (ones * a) / (ones * b)

    def build_axis(lo, size, arr_ref, warr_ref, limit):
        bin_sz = _vdiv(size, jnp.float32(_RH))
        limf = jnp.float32(limit)
        for half in range(2):
            sv = lanes + (half * 16)
            svf = lanes_f + jnp.float32(half * 16)
            t = _f32(lax.shift_right_logical(sv, 1)) + (
                0.25 + 0.5 * _f32(lax.bitwise_and(sv, 1)))
            cs = lo + t * bin_sz
            valid = jnp.logical_and(cs > -1.0, cs < limf)
            cc = jnp.clip(cs, 0.0, limf - 1.0)
            c0 = cc.astype(jnp.int32)
            frac = cc - _f32(c0)
            c1 = jnp.minimum(c0 + 1, limit - 1)
            vf = jnp.where(valid, jnp.float32(0.5), jnp.float32(0.0))
            w0 = (1.0 - frac) * vf
            w1 = frac * vf
            msk = sv < _NS
            pos0 = 2 * sv
            plsc.store_scatter(arr_ref, [pos0], c0, mask=msk)
            plsc.store_scatter(arr_ref, [pos0 + 1], c1, mask=msk)
            plsc.store_scatter(warr_ref, [pos0], w0, mask=msk)
            plsc.store_scatter(warr_ref, [pos0 + 1], w1, mask=msk)

    def roi_body(r, carry):
        rrow = roi_v[r]
        b_i = rrow[0].astype(jnp.int32)
        ax1, ay1 = rrow[1], rrow[2]
        ax2, ay2 = rrow[3], rrow[4]
        bx1, by1 = rrow[6], rrow[7]
        bx2, by2 = rrow[8], rrow[9]
        min_x = jnp.minimum(ax1, bx1)
        min_y = jnp.minimum(ay1, by1)
        max_x = jnp.maximum(ax2, bx2)
        max_y = jnp.maximum(ay2, by2)

        x1s = min_x * _SCALE
        y1s = min_y * _SCALE
        roi_w = jnp.maximum(max_x * _SCALE - x1s, 1.0)
        roi_h = jnp.maximum(max_y * _SCALE - y1s, 1.0)

        build_axis(y1s, roi_h, yarr, wyarr, _H)
        build_axis(x1s, roi_w, xarr, wxarr, _W)

        irow = ipos_v[r]
        xlo0, ylo0, xhi0, yhi0 = irow[0], irow[1], irow[2], irow[3]
        xlo1, ylo1, xhi1, yhi1 = irow[4], irow[5], irow[6], irow[7]

        base_row = b_i * (_H * _W)

        def ph_body(ph, carry2):
            def pix(pw, idx_ref, w_ref, slot):
                ybv = plsc.load_gather(yarr, [4 * ph + qy])
                wyv = plsc.load_gather(wyarr, [4 * ph + qy])
                xbv = plsc.load_gather(xarr, [4 * pw + qx])
                wxv = plsc.load_gather(wxarr, [4 * pw + qx])
                idx16 = base_row + ybv * _W + xbv
                w16 = wyv * wxv
                idx_ref[pl.ds(slot * 16, 16)] = idx16
                w_ref[pl.ds(slot * 16, 16)] = w16

            def build_a(j, c):
                pix(j, idx_a, w_a, j)
                return c

            def build_b(j, c):
                pix(7 + j, idx_b, w_b, j)
                return c

            lax.fori_loop(0, 7, build_a, 0)
            cp_a = pltpu.async_copy(table_hbm.at[idx_a], buf_a, sem_a)
            lax.fori_loop(0, 7, build_b, 0)
            cp_b = pltpu.async_copy(table_hbm.at[idx_b], buf_b, sem_b)

            @pl.when(jnp.logical_or(ph > 0, r > 0))
            def _drain_out():
                pltpu.make_async_copy(out_hbm.at[0, 0], outb, sem_o).wait()

            my0 = jnp.logical_and(ph >= ylo0, ph < yhi0)
            my1 = jnp.logical_and(ph >= ylo1, ph < yhi1)

            def combine(pw, buf, w_ref, slot):
                base = slot * 16
                m0 = _f32(jnp.logical_and(
                    my0, jnp.logical_and(pw >= xlo0, pw < xhi0)))
                m1 = _f32(jnp.logical_and(
                    my1, jnp.logical_and(pw >= xlo1, pw < xhi1)))

                wv = w_ref[pl.ds(base, 16)]
                zero = jnp.zeros((16,), jnp.float32)
                accs = [zero] * (_C // 16)
                for k in range(16):
                    wk = wv[k]
                    for c in range(_C // 16):
                        accs[c] = accs[c] + wk * buf[
                            base + k, pl.ds(c * 16, 16)]
                for c in range(_C // 16):
                    outb[0, pw, pl.ds(c * 16, 16)] = accs[c] * m0
                    outb[1, pw, pl.ds(c * 16, 16)] = accs[c] * m1

            cp_a.wait()

            def comb_a(j, c):
                combine(j, buf_a, w_a, j)
                return c

            lax.fori_loop(0, 7, comb_a, 0)
            cp_b.wait()

            def comb_b(j, c):
                combine(7 + j, buf_b, w_b, j)
                return c

            lax.fori_loop(0, 7, comb_b, 0)

            pltpu.async_copy(outb, out_hbm.at[wid * _RPW + r, ph], sem_o)
            return carry2

        lax.fori_loop(0, _RH, ph_body, 0)
        return carry

    lax.fori_loop(0, _RPW, roi_body, 0)
    pltpu.make_async_copy(out_hbm.at[0, 0], outb, sem_o).wait()


@jax.jit
def _run(table, rois_flat, ipos):
    mesh = plsc.VectorSubcoreMesh(core_axis_name="c", subcore_axis_name="s")
    fn = functools.partial(
        pl.kernel,
        out_type=jax.ShapeDtypeStruct(
            (_N_ROIS, _RH, _NUM_INST, _RW, _C), jnp.float32),
        mesh=mesh,
        scratch_types=[
            pltpu.VMEM((_RPW, 16), jnp.float32),
            pltpu.VMEM((_RPW, 16), jnp.int32),
            pltpu.VMEM((64,), jnp.int32),
            pltpu.VMEM((64,), jnp.int32),
            pltpu.VMEM((64,), jnp.float32),
            pltpu.VMEM((64,), jnp.float32),
            pltpu.VMEM((112,), jnp.int32),
            pltpu.VMEM((112,), jnp.int32),
            pltpu.VMEM((112,), jnp.float32),
            pltpu.VMEM((112,), jnp.float32),
            pltpu.VMEM((112, _C), jnp.float32),
            pltpu.VMEM((112, _C), jnp.float32),
            pltpu.VMEM((_NUM_INST, _RW, _C), jnp.float32),
            pltpu.SemaphoreType.DMA,
            pltpu.SemaphoreType.DMA,
            pltpu.SemaphoreType.DMA,
        ],
        compiler_params=pltpu.CompilerParams(needs_layout_passes=False),
    )(_sc_body)
    return fn(table, rois_flat, ipos)


def kernel(feature_maps, rois):
    table = jnp.transpose(feature_maps, (0, 2, 3, 1)).reshape(
        _NB * _H * _W, _C)
    rois_flat = rois.reshape(_N_ROIS, _NUM_INST * 5)
    rois_flat = jnp.pad(rois_flat, ((0, 0), (0, 6)))
    min_x = rois[:, :, 1].min(axis=1)
    min_y = rois[:, :, 2].min(axis=1)
    max_x = rois[:, :, 3].max(axis=1)
    max_y = rois[:, :, 4].max(axis=1)
    brois_w = max_x - min_x
    brois_h = max_y - min_y
    h_ratio = _RH / brois_h
    w_ratio = _RW / brois_w
    ratios = jnp.stack(
        [w_ratio, h_ratio, w_ratio, h_ratio], axis=1).reshape(-1, 1, 4)
    bounding_lt = jnp.tile(
        jnp.stack([min_x, min_y], axis=1)[:, None, :], (1, 1, 2))
    ins_pos = ((rois[:, :, 1:] - bounding_lt) * ratios).astype(jnp.int32)
    ipos = jnp.pad(ins_pos.reshape(_N_ROIS, 8), ((0, 0), (0, 8)))
    out5 = _run(table, rois_flat, ipos)
    return jnp.transpose(out5, (0, 2, 4, 1, 3)).reshape(
        _N_ROIS, _NUM_INST * _C, _RH, _RW)
